# baseline (speedup 1.0000x reference)
import functools

import jax
import jax.numpy as jnp
from jax import lax
from jax.experimental import pallas as pl
from jax.experimental.pallas import tpu as pltpu
from jax.experimental.pallas import tpu_sc as plsc

B = 16384
IN = 118
K = 1024
D = 64
DP = 128
NG = 2
COMMIT = 0.25

BB = 1024
NB = B // BB

NC = 2
NS = 16
NW = NC * NS
BPW = B // NW
CHUNK = 128
NCH = BPW // CHUNK


def _encode_body(x_ref, w1, b1, w2, b2, w3, b3, w4, b4, et_ref, esq_ref,
                 ze_ref, idx_ref):
    x = x_ref[...]
    h = jnp.tanh(jnp.dot(x, w1[...]) + b1[...])
    h = jnp.tanh(jnp.dot(h, w2[...]) + b2[...])
    h = jnp.tanh(jnp.dot(h, w3[...]) + b3[...])
    ze = jnp.dot(h, w4[...]) + b4[...]
    ze_ref[...] = ze
    m = jnp.dot(ze, et_ref[...])
    zsq = jnp.sum(ze * ze, axis=1, keepdims=True)
    dist = (zsq + esq_ref[...]) - 2.0 * m
    dmin = jnp.min(dist, axis=1, keepdims=True)
    iota = lax.broadcasted_iota(jnp.int32, dist.shape, 1)
    idx = jnp.min(jnp.where(dist == dmin, iota, jnp.int32(K)), axis=1)
    idx_ref[0, 0, :] = idx


def _encode(x, w1t, b1, w2t, b2, w3t, b3, w4t, b4, et, esq):
    full = lambda *s: pl.BlockSpec(s, lambda i: (0,) * len(s))
    return pl.pallas_call(
        _encode_body,
        grid=(NB,),
        in_specs=[
            pl.BlockSpec((BB, IN), lambda i: (i, 0)),
            full(IN, 60), full(1, 60),
            full(60, 30), full(1, 30),
            full(30, 10), full(1, 10),
            full(10, D), full(1, D),
            full(D, K), full(1, K),
        ],
        out_specs=[
            pl.BlockSpec((BB, D), lambda i: (i, 0)),
            pl.BlockSpec((1, 1, BB), lambda i: (i, 0, 0)),
        ],
        out_shape=[
            jax.ShapeDtypeStruct((B, D), jnp.float32),
            jax.ShapeDtypeStruct((NB, 1, BB), jnp.int32),
        ],
    )(x, w1t, b1, w2t, b2, w3t, b3, w4t, b4, et, esq)


_sc_gather_impl = None


def _build_sc_gather():
    mesh = plsc.VectorSubcoreMesh(core_axis_name="c", subcore_axis_name="s",
                                  num_cores=NC, num_subcores=NS)

    @functools.partial(
        pl.kernel,
        mesh=mesh,
        out_type=jax.ShapeDtypeStruct((B, DP), jnp.float32),
        scratch_types=[
            pltpu.VMEM((NCH, CHUNK), jnp.int32),
            pltpu.VMEM((BPW, DP), jnp.float32),
            pltpu.SemaphoreType.DMA,
        ],
    )
    def body(table_hbm, idx_hbm, out_hbm, idx_v, rows_v, sem):
        wid = lax.axis_index("s") * NC + lax.axis_index("c")
        base = wid * BPW
        pltpu.sync_copy(idx_hbm.at[wid], idx_v)
        copies = [
            pltpu.async_copy(table_hbm.at[idx_v.at[j]],
                             rows_v.at[pl.ds(j * CHUNK, CHUNK)], sem)
            for j in range(NCH)
        ]
        for c in copies:
            c.wait()
        pltpu.sync_copy(rows_v, out_hbm.at[pl.ds(base, BPW)])

    return body


def _sc_gather(table, idx):
    global _sc_gather_impl
    if _sc_gather_impl is None:
        _sc_gather_impl = _build_sc_gather()
    return _sc_gather_impl(table, idx)


def _decode_body(x_ref, ze_ref, zqp_ref, w5, b5, w6, b6, w7, b7, w8, b8,
                 w9, b9, w10, b10, zq_ref, xhat_ref, zaug_ref, gamma_ref,
                 vq_ref):
    zq = zqp_ref[:, :D]
    zq_ref[...] = zq
    h = jnp.tanh(jnp.dot(zq, w5[...]) + b5[...])
    h = jnp.tanh(jnp.dot(h, w6[...]) + b6[...])
    h = jnp.tanh(jnp.dot(h, w7[...]) + b7[...])
    xh = jnp.dot(h, w8[...]) + b8[...]
    xhat_ref[...] = xh
    x = x_ref[...]
    diff = x - xh
    dn = jnp.sqrt(jnp.sum(diff * diff, axis=1, keepdims=True))
    xn = jnp.sqrt(jnp.sum(x * x, axis=1, keepdims=True))
    xhn = jnp.sqrt(jnp.sum(xh * xh, axis=1, keepdims=True))
    rec1 = dn / (xn + 1e-12)
    rec2 = jnp.sum(x * xh, axis=1, keepdims=True) / (
        jnp.maximum(xn, 1e-8) * jnp.maximum(xhn, 1e-8))
    zaug = jnp.concatenate([zq, rec1, rec2], axis=1)
    zaug_ref[...] = zaug
    h9 = jnp.tanh(jnp.dot(zaug, w9[...]) + b9[...])
    logits = jnp.dot(h9, w10[...]) + b10[...]
    lmax = jnp.max(logits, axis=1, keepdims=True)
    e = jnp.exp(logits - lmax)
    gamma_ref[...] = e / jnp.sum(e, axis=1, keepdims=True)
    d = zq - ze_ref[...]
    vq_ref[0, 0, 0] = jnp.sum(d * d)


def _decode(x, ze, zq, w5t, b5, w6t, b6, w7t, b7, w8t, b8, w9t, b9,
            w10t, b10):
    full = lambda *s: pl.BlockSpec(s, lambda i: (0,) * len(s))
    return pl.pallas_call(
        _decode_body,
        grid=(NB,),
        in_specs=[
            pl.BlockSpec((BB, IN), lambda i: (i, 0)),
            pl.BlockSpec((BB, D), lambda i: (i, 0)),
            pl.BlockSpec((BB, DP), lambda i: (i, 0)),
            full(D, 10), full(1, 10),
            full(10, 30), full(1, 30),
            full(30, 60), full(1, 60),
            full(60, IN), full(1, IN),
            full(D + 2, 10), full(1, 10),
            full(10, NG), full(1, NG),
        ],
        out_specs=[
            pl.BlockSpec((BB, D), lambda i: (i, 0)),
            pl.BlockSpec((BB, IN), lambda i: (i, 0)),
            pl.BlockSpec((BB, D + 2), lambda i: (i, 0)),
            pl.BlockSpec((BB, NG), lambda i: (i, 0)),
            pl.BlockSpec((1, 1, 1), lambda i: (i, 0, 0),
                         memory_space=pltpu.SMEM),
        ],
        out_shape=[
            jax.ShapeDtypeStruct((B, D), jnp.float32),
            jax.ShapeDtypeStruct((B, IN), jnp.float32),
            jax.ShapeDtypeStruct((B, D + 2), jnp.float32),
            jax.ShapeDtypeStruct((B, NG), jnp.float32),
            jax.ShapeDtypeStruct((NB, 1, 1), jnp.float32),
        ],
    )(x, ze, zq, w5t, b5, w6t, b6, w7t, b7, w8t, b8, w9t, b9, w10t, b10)


def kernel(x, params):
    p = params
    row = lambda b: b.reshape(1, -1)
    et = p['codebook'].T
    esq = row(jnp.sum(p['codebook'] ** 2, axis=1))

    ze, idx3 = _encode(
        x, p['W1'].T, row(p['b1']), p['W2'].T, row(p['b2']),
        p['W3'].T, row(p['b3']), p['W4'].T, row(p['b4']), et, esq)

    idx = idx3.reshape(NW, NCH, CHUNK)
    table_pad = jnp.pad(p['codebook'], ((0, 0), (0, DP - D)))
    zqp = _sc_gather(table_pad, idx)

    zq, xhat, zaug, gamma, vq_parts = _decode(
        x, ze, zqp, p['W5'].T, row(p['b5']), p['W6'].T, row(p['b6']),
        p['W7'].T, row(p['b7']), p['W8'].T, row(p['b8']),
        p['W9'].T, row(p['b9']), p['W10'].T, row(p['b10']))

    m = jnp.sum(vq_parts) / (B * D)
    vq_loss = m + COMMIT * m
    return (ze, zq, vq_loss, xhat, zaug, gamma)


# speedup vs baseline: 4.0638x; 4.0638x over previous
import functools

import jax
import jax.numpy as jnp
from jax import lax
from jax.experimental import pallas as pl
from jax.experimental.pallas import tpu as pltpu
from jax.experimental.pallas import tpu_sc as plsc

B = 16384
IN = 118
K = 1024
D = 64
NG = 2
COMMIT = 0.25

BB = 1024
NB = B // BB

NC = 2
NS = 16
NW = NC * NS
BPW = B // NW


def _encode_body(x_ref, w1, b1, w2, b2, w3, b3, w4, b4, et_ref, esq_ref,
                 ze_ref, idx_ref):
    x = x_ref[...]
    h = jnp.tanh(jnp.dot(x, w1[...]) + b1[...])
    h = jnp.tanh(jnp.dot(h, w2[...]) + b2[...])
    h = jnp.tanh(jnp.dot(h, w3[...]) + b3[...])
    ze = jnp.dot(h, w4[...]) + b4[...]
    ze_ref[...] = ze
    m = jnp.dot(ze, et_ref[...])
    zsq = jnp.sum(ze * ze, axis=1, keepdims=True)
    dist = (zsq + esq_ref[...]) - 2.0 * m
    dmin = jnp.min(dist, axis=1, keepdims=True)
    iota = lax.broadcasted_iota(jnp.int32, dist.shape, 1)
    idx = jnp.min(jnp.where(dist == dmin, iota, jnp.int32(K)), axis=1)
    idx_ref[0, 0, :] = idx


def _encode(x, w1t, b1, w2t, b2, w3t, b3, w4t, b4, et, esq):
    full = lambda *s: pl.BlockSpec(s, lambda i: (0,) * len(s))
    return pl.pallas_call(
        _encode_body,
        grid=(NB,),
        in_specs=[
            pl.BlockSpec((BB, IN), lambda i: (i, 0)),
            full(IN, 60), full(1, 60),
            full(60, 30), full(1, 30),
            full(30, 10), full(1, 10),
            full(10, D), full(1, D),
            full(D, K), full(1, K),
        ],
        out_specs=[
            pl.BlockSpec((BB, D), lambda i: (i, 0)),
            pl.BlockSpec((1, 1, BB), lambda i: (i, 0, 0)),
        ],
        out_shape=[
            jax.ShapeDtypeStruct((B, D), jnp.float32),
            jax.ShapeDtypeStruct((NB, 1, BB), jnp.int32),
        ],
    )(x, w1t, b1, w2t, b2, w3t, b3, w4t, b4, et, esq)


_sc_gather_impl = None


def _build_sc_gather():
    mesh = plsc.VectorSubcoreMesh(core_axis_name="c", subcore_axis_name="s",
                                  num_cores=NC, num_subcores=NS)
    L = 16
    RPB = L
    NBLK = BPW // RPB

    @functools.partial(
        pl.kernel,
        mesh=mesh,
        compiler_params=pltpu.CompilerParams(needs_layout_passes=False),
        out_type=jax.ShapeDtypeStruct((B * D,), jnp.float32),
        scratch_types=[
            pltpu.VMEM((K * D,), jnp.float32),
            pltpu.VMEM((BPW,), jnp.int32),
            pltpu.VMEM((BPW * D,), jnp.float32),
        ],
    )
    def body(table_hbm, idx_hbm, out_hbm, table_v, idx_v, rows_v):
        wid = lax.axis_index("s") * NC + lax.axis_index("c")
        base = wid * BPW
        pltpu.sync_copy(table_hbm, table_v)
        pltpu.sync_copy(idx_hbm.at[pl.ds(base, BPW)], idx_v)
        lane = lax.broadcasted_iota(jnp.int32, (L,), 0)

        def blk(t, _):
            idxv = idx_v[pl.ds(t * RPB, RPB)]
            src0 = idxv * D
            dst0 = (t * RPB + lane) * D
            for j in range(D):
                col = plsc.load_gather(table_v, [src0 + j])
                plsc.store_scatter(rows_v, [dst0 + j], col)
            return _

        lax.fori_loop(0, NBLK, blk, 0)
        pltpu.sync_copy(rows_v, out_hbm.at[pl.ds(base * D, BPW * D)])

    return body


def _sc_gather(table, idx):
    global _sc_gather_impl
    if _sc_gather_impl is None:
        _sc_gather_impl = _build_sc_gather()
    return _sc_gather_impl(table, idx)


def _decode_body(x_ref, ze_ref, zq_ref, w5, b5, w6, b6, w7, b7, w8, b8,
                 w9, b9, w10, b10, xhat_ref, zaug_ref, gamma_ref, vq_ref):
    zq = zq_ref[...]
    h = jnp.tanh(jnp.dot(zq, w5[...]) + b5[...])
    h = jnp.tanh(jnp.dot(h, w6[...]) + b6[...])
    h = jnp.tanh(jnp.dot(h, w7[...]) + b7[...])
    xh = jnp.dot(h, w8[...]) + b8[...]
    xhat_ref[...] = xh
    x = x_ref[...]
    diff = x - xh
    dn = jnp.sqrt(jnp.sum(diff * diff, axis=1, keepdims=True))
    xn = jnp.sqrt(jnp.sum(x * x, axis=1, keepdims=True))
    xhn = jnp.sqrt(jnp.sum(xh * xh, axis=1, keepdims=True))
    rec1 = dn / (xn + 1e-12)
    rec2 = jnp.sum(x * xh, axis=1, keepdims=True) / (
        jnp.maximum(xn, 1e-8) * jnp.maximum(xhn, 1e-8))
    zaug = jnp.concatenate([zq, rec1, rec2], axis=1)
    zaug_ref[...] = zaug
    h9 = jnp.tanh(jnp.dot(zaug, w9[...]) + b9[...])
    logits = jnp.dot(h9, w10[...]) + b10[...]
    lmax = jnp.max(logits, axis=1, keepdims=True)
    e = jnp.exp(logits - lmax)
    gamma_ref[...] = e / jnp.sum(e, axis=1, keepdims=True)
    d = zq - ze_ref[...]
    vq_ref[0, 0, 0] = jnp.sum(d * d)


def _decode(x, ze, zq, w5t, b5, w6t, b6, w7t, b7, w8t, b8, w9t, b9,
            w10t, b10):
    full = lambda *s: pl.BlockSpec(s, lambda i: (0,) * len(s))
    return pl.pallas_call(
        _decode_body,
        grid=(NB,),
        in_specs=[
            pl.BlockSpec((BB, IN), lambda i: (i, 0)),
            pl.BlockSpec((BB, D), lambda i: (i, 0)),
            pl.BlockSpec((BB, D), lambda i: (i, 0)),
            full(D, 10), full(1, 10),
            full(10, 30), full(1, 30),
            full(30, 60), full(1, 60),
            full(60, IN), full(1, IN),
            full(D + 2, 10), full(1, 10),
            full(10, NG), full(1, NG),
        ],
        out_specs=[
            pl.BlockSpec((BB, IN), lambda i: (i, 0)),
            pl.BlockSpec((BB, D + 2), lambda i: (i, 0)),
            pl.BlockSpec((BB, NG), lambda i: (i, 0)),
            pl.BlockSpec((1, 1, 1), lambda i: (i, 0, 0),
                         memory_space=pltpu.SMEM),
        ],
        out_shape=[
            jax.ShapeDtypeStruct((B, IN), jnp.float32),
            jax.ShapeDtypeStruct((B, D + 2), jnp.float32),
            jax.ShapeDtypeStruct((B, NG), jnp.float32),
            jax.ShapeDtypeStruct((NB, 1, 1), jnp.float32),
        ],
    )(x, ze, zq, w5t, b5, w6t, b6, w7t, b7, w8t, b8, w9t, b9, w10t, b10)


def kernel(x, params):
    p = params
    row = lambda b: b.reshape(1, -1)
    et = p['codebook'].T
    esq = row(jnp.sum(p['codebook'] ** 2, axis=1))

    ze, idx3 = _encode(
        x, p['W1'].T, row(p['b1']), p['W2'].T, row(p['b2']),
        p['W3'].T, row(p['b3']), p['W4'].T, row(p['b4']), et, esq)

    idx = idx3.reshape(B)
    zq = _sc_gather(p['codebook'].reshape(K * D), idx).reshape(B, D)

    xhat, zaug, gamma, vq_parts = _decode(
        x, ze, zq, p['W5'].T, row(p['b5']), p['W6'].T, row(p['b6']),
        p['W7'].T, row(p['b7']), p['W8'].T, row(p['b8']),
        p['W9'].T, row(p['b9']), p['W10'].T, row(p['b10']))

    m = jnp.sum(vq_parts) / (B * D)
    vq_loss = m + COMMIT * m
    return (ze, zq, vq_loss, xhat, zaug, gamma)


# speedup vs baseline: 6.3859x; 1.5714x over previous
import functools

import jax
import jax.numpy as jnp
from jax import lax
from jax.experimental import pallas as pl
from jax.experimental.pallas import tpu as pltpu
from jax.experimental.pallas import tpu_sc as plsc

B = 16384
IN = 118
K = 1024
D = 64
NG = 2
COMMIT = 0.25

BBE = 2048
NBE = B // BBE

NC = 2
NS = 16
NW = NC * NS
BPW = B // NW
L = 16


def _encode_body(xt_ref, w1, b1, w2, b2, w3, b3, w4, b4, et_ref, esq_ref,
                 zet_ref, idx_ref):
    xt = xt_ref[...]
    h = jnp.tanh(jnp.dot(w1[...], xt) + b1[...])
    h = jnp.tanh(jnp.dot(w2[...], h) + b2[...])
    h = jnp.tanh(jnp.dot(w3[...], h) + b3[...])
    zet = jnp.dot(w4[...], h) + b4[...]
    zet_ref[...] = zet
    m2 = lax.dot_general(et_ref[...], -2.0 * zet,
                         (((0,), (0,)), ((), ())))
    zsq = jnp.sum(zet * zet, axis=0, keepdims=True)
    dist = (zsq + esq_ref[...]) + m2
    dmin = jnp.min(dist, axis=0, keepdims=True)
    iota = lax.broadcasted_iota(jnp.int32, dist.shape, 0)
    idx = jnp.min(jnp.where(dist == dmin, iota, jnp.int32(K)), axis=0,
                  keepdims=True)
    idx_ref[...] = idx


def _encode(xt, w1, b1, w2, b2, w3, b3, w4, b4, et, esq):
    full = lambda *s: pl.BlockSpec(s, lambda i: (0,) * len(s))
    return pl.pallas_call(
        _encode_body,
        grid=(NBE,),
        in_specs=[
            pl.BlockSpec((IN, BBE), lambda i: (0, i)),
            full(60, IN), full(60, 1),
            full(30, 60), full(30, 1),
            full(10, 30), full(10, 1),
            full(D, 10), full(D, 1),
            full(D, K), full(K, 1),
        ],
        out_specs=[
            pl.BlockSpec((D, BBE), lambda i: (0, i)),
            pl.BlockSpec((1, BBE), lambda i: (0, i)),
        ],
        out_shape=[
            jax.ShapeDtypeStruct((D, B), jnp.float32),
            jax.ShapeDtypeStruct((1, B), jnp.int32),
        ],
    )(xt, w1, b1, w2, b2, w3, b3, w4, b4, et, esq)


_sc_gather_impl = None


def _build_sc_gather():
    mesh = plsc.VectorSubcoreMesh(core_axis_name="c", subcore_axis_name="s",
                                  num_cores=NC, num_subcores=NS)
    NG16 = BPW // L

    @functools.partial(
        pl.kernel,
        mesh=mesh,
        compiler_params=pltpu.CompilerParams(needs_layout_passes=False),
        out_type=jax.ShapeDtypeStruct((B * D,), jnp.float32),
        scratch_types=[
            pltpu.VMEM((D * K,), jnp.float32),
            pltpu.VMEM((BPW,), jnp.int32),
            pltpu.VMEM((D * BPW,), jnp.float32),
        ],
    )
    def body(et_hbm, idx_hbm, out_hbm, et_v, idx_v, zqt_v):
        wid = lax.axis_index("s") * NC + lax.axis_index("c")
        base = wid * BPW
        pltpu.sync_copy(et_hbm, et_v)
        pltpu.sync_copy(idx_hbm.at[pl.ds(base, BPW)], idx_v)

        def grp_body(g, carry):
            idxv = idx_v[pl.ds(g * L, L)]
            for d in range(D):
                col = plsc.load_gather(et_v, [idxv + d * K])
                zqt_v[pl.ds(d * BPW + g * L, L)] = col
            return carry

        lax.fori_loop(0, NG16, grp_body, 0)
        pltpu.sync_copy(zqt_v, out_hbm.at[pl.ds(base * D, BPW * D)])

    return body


def _sc_gather(et_flat, idx):
    global _sc_gather_impl
    if _sc_gather_impl is None:
        _sc_gather_impl = _build_sc_gather()
    return _sc_gather_impl(et_flat, idx)


def _decode_body(xt_ref, zet_ref, zqc_ref, w5, b5, w6, b6, w7, b7, w8, b8,
                 w9, b9, w10, b10, zqt_ref, xht_ref, zaugt_ref, gt_ref,
                 vq_ref):
    zqt = zqc_ref[0]
    zqt_ref[...] = zqt
    h = jnp.tanh(jnp.dot(w5[...], zqt) + b5[...])
    h = jnp.tanh(jnp.dot(w6[...], h) + b6[...])
    h = jnp.tanh(jnp.dot(w7[...], h) + b7[...])
    xht = jnp.dot(w8[...], h) + b8[...]
    xht_ref[...] = xht
    xt = xt_ref[...]
    diff = xt - xht
    dn = jnp.sqrt(jnp.sum(diff * diff, axis=0, keepdims=True))
    xn = jnp.sqrt(jnp.sum(xt * xt, axis=0, keepdims=True))
    xhn = jnp.sqrt(jnp.sum(xht * xht, axis=0, keepdims=True))
    rec1 = dn / (xn + 1e-12)
    rec2 = jnp.sum(xt * xht, axis=0, keepdims=True) / (
        jnp.maximum(xn, 1e-8) * jnp.maximum(xhn, 1e-8))
    zaugt = jnp.concatenate([zqt, rec1, rec2], axis=0)
    zaugt_ref[...] = zaugt
    h9 = jnp.tanh(jnp.dot(w9[...], zaugt) + b9[...])
    logits = jnp.dot(w10[...], h9) + b10[...]
    lmax = jnp.max(logits, axis=0, keepdims=True)
    e = jnp.exp(logits - lmax)
    gt_ref[...] = e / jnp.sum(e, axis=0, keepdims=True)
    d = zqt - zet_ref[...]
    vq_ref[0, 0, 0] = jnp.sum(d * d)


def _decode(xt, zet, zqc, w5, b5, w6, b6, w7, b7, w8, b8, w9, b9,
            w10, b10):
    full = lambda *s: pl.BlockSpec(s, lambda i: (0,) * len(s))
    return pl.pallas_call(
        _decode_body,
        grid=(NW,),
        in_specs=[
            pl.BlockSpec((IN, BPW), lambda i: (0, i)),
            pl.BlockSpec((D, BPW), lambda i: (0, i)),
            pl.BlockSpec((1, D, BPW), lambda i: (i, 0, 0)),
            full(10, D), full(10, 1),
            full(30, 10), full(30, 1),
            full(60, 30), full(60, 1),
            full(IN, 60), full(IN, 1),
            full(10, D + 2), full(10, 1),
            full(NG, 10), full(NG, 1),
        ],
        out_specs=[
            pl.BlockSpec((D, BPW), lambda i: (0, i)),
            pl.BlockSpec((IN, BPW), lambda i: (0, i)),
            pl.BlockSpec((D + 2, BPW), lambda i: (0, i)),
            pl.BlockSpec((NG, BPW), lambda i: (0, i)),
            pl.BlockSpec((1, 1, 1), lambda i: (i, 0, 0),
                         memory_space=pltpu.SMEM),
        ],
        out_shape=[
            jax.ShapeDtypeStruct((D, B), jnp.float32),
            jax.ShapeDtypeStruct((IN, B), jnp.float32),
            jax.ShapeDtypeStruct((D + 2, B), jnp.float32),
            jax.ShapeDtypeStruct((NG, B), jnp.float32),
            jax.ShapeDtypeStruct((NW, 1, 1), jnp.float32),
        ],
    )(xt, zet, zqc, w5, b5, w6, b6, w7, b7, w8, b8, w9, b9, w10, b10)


def kernel(x, params):
    p = params
    col = lambda b: b.reshape(-1, 1)
    xt = x.T
    et = p['codebook'].T
    esq = jnp.sum(p['codebook'] ** 2, axis=1).reshape(K, 1)

    zet, idx2 = _encode(
        xt, p['W1'], col(p['b1']), p['W2'], col(p['b2']),
        p['W3'], col(p['b3']), p['W4'], col(p['b4']), et, esq)

    zq_flat = _sc_gather(et.reshape(D * K), idx2.reshape(B))
    zqc = zq_flat.reshape(NW, D, BPW)

    zqt, xht, zaugt, gt, vq_parts = _decode(
        xt, zet, zqc, p['W5'], col(p['b5']), p['W6'], col(p['b6']),
        p['W7'], col(p['b7']), p['W8'], col(p['b8']),
        p['W9'], col(p['b9']), p['W10'], col(p['b10']))

    m = jnp.sum(vq_parts) / (B * D)
    vq_loss = m + COMMIT * m
    return (zet.T, zqt.T, vq_loss, xht.T, zaugt.T, gt.T)


# speedup vs baseline: 8.5160x; 1.3336x over previous
import functools

import jax
import jax.numpy as jnp
from jax import lax
from jax.experimental import pallas as pl
from jax.experimental.pallas import tpu as pltpu
from jax.experimental.pallas import tpu_sc as plsc

B = 16384
IN = 118
K = 1024
D = 64
NG = 2
COMMIT = 0.25

BBE = 4096
NBE = B // BBE
BBD = 2048
NBD = B // BBD

NC = 2
NS = 16
NW = NC * NS
BPW = B // NW
L = 16


def _encode_body(xt_ref, w1, b1, w2, b2, w3, b3, w4, b4, et_ref, esq_ref,
                 zet_ref, idx_ref):
    xt = xt_ref[...]
    h = jnp.tanh(jnp.dot(w1[...], xt) + b1[...])
    h = jnp.tanh(jnp.dot(w2[...], h) + b2[...])
    h = jnp.tanh(jnp.dot(w3[...], h) + b3[...])
    zet = jnp.dot(w4[...], h) + b4[...]
    zet_ref[...] = zet
    m2 = lax.dot_general(et_ref[...], -2.0 * zet,
                         (((0,), (0,)), ((), ())))
    zsq = jnp.sum(zet * zet, axis=0, keepdims=True)
    dist = (zsq + esq_ref[...]) + m2
    dmin = jnp.min(dist, axis=0, keepdims=True)
    iota = lax.broadcasted_iota(jnp.int32, dist.shape, 0)
    idx = jnp.min(jnp.where(dist == dmin, iota, jnp.int32(K)), axis=0,
                  keepdims=True)
    idx_ref[...] = idx


def _encode(xt, w1, b1, w2, b2, w3, b3, w4, b4, et, esq):
    full = lambda *s: pl.BlockSpec(s, lambda i: (0,) * len(s))
    return pl.pallas_call(
        _encode_body,
        grid=(NBE,),
        in_specs=[
            pl.BlockSpec((IN, BBE), lambda i: (0, i)),
            full(60, IN), full(60, 1),
            full(30, 60), full(30, 1),
            full(10, 30), full(10, 1),
            full(D, 10), full(D, 1),
            full(D, K), full(K, 1),
        ],
        out_specs=[
            pl.BlockSpec((D, BBE), lambda i: (0, i)),
            pl.BlockSpec((1, BBE), lambda i: (0, i)),
        ],
        out_shape=[
            jax.ShapeDtypeStruct((D, B), jnp.float32),
            jax.ShapeDtypeStruct((1, B), jnp.int32),
        ],
    )(xt, w1, b1, w2, b2, w3, b3, w4, b4, et, esq)


_sc_gather_impl = None


def _build_sc_gather():
    mesh = plsc.VectorSubcoreMesh(core_axis_name="c", subcore_axis_name="s",
                                  num_cores=NC, num_subcores=NS)
    NG16 = BPW // L

    @functools.partial(
        pl.kernel,
        mesh=mesh,
        compiler_params=pltpu.CompilerParams(needs_layout_passes=False),
        out_type=jax.ShapeDtypeStruct((D, B), jnp.float32),
        scratch_types=[
            pltpu.VMEM((D * K,), jnp.float32),
            pltpu.VMEM((BPW,), jnp.int32),
            pltpu.VMEM((D, BPW), jnp.float32),
        ],
    )
    def body(et_hbm, idx_hbm, out_hbm, et_v, idx_v, zqt_v):
        wid = lax.axis_index("s") * NC + lax.axis_index("c")
        base = wid * BPW
        pltpu.sync_copy(et_hbm, et_v)
        pltpu.sync_copy(idx_hbm.at[pl.ds(base, BPW)], idx_v)

        def grp_body(g, carry):
            idxv = idx_v[pl.ds(g * L, L)]
            for d in range(D):
                col = plsc.load_gather(et_v, [idxv + d * K])
                zqt_v[d, pl.ds(g * L, L)] = col
            return carry

        lax.fori_loop(0, NG16, grp_body, 0)
        pltpu.sync_copy(zqt_v, out_hbm.at[:, pl.ds(base, BPW)])

    return body


def _sc_gather(et_flat, idx):
    global _sc_gather_impl
    if _sc_gather_impl is None:
        _sc_gather_impl = _build_sc_gather()
    return _sc_gather_impl(et_flat, idx)


def _decode_body(xt_ref, zet_ref, zqt_in_ref, w5, b5, w6, b6, w7, b7,
                 w8, b8, w9, b9, w10, b10, xht_ref, zaugt_ref, gt_ref,
                 vq_ref):
    zqt = zqt_in_ref[...]
    h = jnp.tanh(jnp.dot(w5[...], zqt) + b5[...])
    h = jnp.tanh(jnp.dot(w6[...], h) + b6[...])
    h = jnp.tanh(jnp.dot(w7[...], h) + b7[...])
    xht = jnp.dot(w8[...], h) + b8[...]
    xht_ref[...] = xht
    xt = xt_ref[...]
    diff = xt - xht
    dn = jnp.sqrt(jnp.sum(diff * diff, axis=0, keepdims=True))
    xn = jnp.sqrt(jnp.sum(xt * xt, axis=0, keepdims=True))
    xhn = jnp.sqrt(jnp.sum(xht * xht, axis=0, keepdims=True))
    rec1 = dn / (xn + 1e-12)
    rec2 = jnp.sum(xt * xht, axis=0, keepdims=True) / (
        jnp.maximum(xn, 1e-8) * jnp.maximum(xhn, 1e-8))
    zaugt = jnp.concatenate([zqt, rec1, rec2], axis=0)
    zaugt_ref[...] = zaugt
    h9 = jnp.tanh(jnp.dot(w9[...], zaugt) + b9[...])
    logits = jnp.dot(w10[...], h9) + b10[...]
    lmax = jnp.max(logits, axis=0, keepdims=True)
    e = jnp.exp(logits - lmax)
    gt_ref[...] = e / jnp.sum(e, axis=0, keepdims=True)
    d = zqt - zet_ref[...]
    vq_ref[0, 0, 0] = jnp.sum(d * d)


def _decode(xt, zet, zqt, w5, b5, w6, b6, w7, b7, w8, b8, w9, b9,
            w10, b10):
    full = lambda *s: pl.BlockSpec(s, lambda i: (0,) * len(s))
    return pl.pallas_call(
        _decode_body,
        grid=(NBD,),
        in_specs=[
            pl.BlockSpec((IN, BBD), lambda i: (0, i)),
            pl.BlockSpec((D, BBD), lambda i: (0, i)),
            pl.BlockSpec((D, BBD), lambda i: (0, i)),
            full(10, D), full(10, 1),
            full(30, 10), full(30, 1),
            full(60, 30), full(60, 1),
            full(IN, 60), full(IN, 1),
            full(10, D + 2), full(10, 1),
            full(NG, 10), full(NG, 1),
        ],
        out_specs=[
            pl.BlockSpec((IN, BBD), lambda i: (0, i)),
            pl.BlockSpec((D + 2, BBD), lambda i: (0, i)),
            pl.BlockSpec((NG, BBD), lambda i: (0, i)),
            pl.BlockSpec((1, 1, 1), lambda i: (i, 0, 0),
                         memory_space=pltpu.SMEM),
        ],
        out_shape=[
            jax.ShapeDtypeStruct((IN, B), jnp.float32),
            jax.ShapeDtypeStruct((D + 2, B), jnp.float32),
            jax.ShapeDtypeStruct((NG, B), jnp.float32),
            jax.ShapeDtypeStruct((NBD, 1, 1), jnp.float32),
        ],
    )(xt, zet, zqt, w5, b5, w6, b6, w7, b7, w8, b8, w9, b9, w10, b10)


def kernel(x, params):
    p = params
    col = lambda b: b.reshape(-1, 1)
    xt = x.T
    et = p['codebook'].T
    esq = jnp.sum(p['codebook'] ** 2, axis=1).reshape(K, 1)

    zet, idx2 = _encode(
        xt, p['W1'], col(p['b1']), p['W2'], col(p['b2']),
        p['W3'], col(p['b3']), p['W4'], col(p['b4']), et, esq)

    zqt = _sc_gather(et.reshape(D * K), idx2.reshape(B))

    xht, zaugt, gt, vq_parts = _decode(
        xt, zet, zqt, p['W5'], col(p['b5']), p['W6'], col(p['b6']),
        p['W7'], col(p['b7']), p['W8'], col(p['b8']),
        p['W9'], col(p['b9']), p['W10'], col(p['b10']))

    m = jnp.sum(vq_parts) / (B * D)
    vq_loss = m + COMMIT * m
    return (zet.T, zqt.T, vq_loss, xht.T, zaugt.T, gt.T)


# speedup vs baseline: 9.5957x; 1.1268x over previous
import functools

import jax
import jax.numpy as jnp
from jax import lax
from jax.experimental import pallas as pl
from jax.experimental.pallas import tpu as pltpu
from jax.experimental.pallas import tpu_sc as plsc

B = 16384
IN = 118
K = 1024
D = 64
NG = 2
COMMIT = 0.25

BBE = 4096
NBE = B // BBE
BBD = 2048
NBD = B // BBD

NC = 2
NS = 16
NW = NC * NS
BPW = B // NW
L = 16


def _encode_body(xt_ref, w1, b1, w2, b2, w3, b3, w4t, b4, et_ref, esq_ref,
                 zet_ref, idx_ref):
    xt = xt_ref[...]
    h = jnp.tanh(jnp.dot(w1[...], xt) + b1[...])
    h = jnp.tanh(jnp.dot(w2[...], h) + b2[...])
    h = jnp.tanh(jnp.dot(w3[...], h) + b3[...])
    zet = lax.dot_general(w4t[...], h, (((0,), (0,)), ((), ()))) + b4[...]
    zet_ref[...] = zet
    m2 = lax.dot_general(et_ref[...], -2.0 * zet,
                         (((0,), (0,)), ((), ())))
    zsq = jnp.sum(zet * zet, axis=0, keepdims=True)
    dist = (zsq + esq_ref[...]) + m2
    dmin = jnp.min(dist, axis=0, keepdims=True)
    iota = lax.broadcasted_iota(jnp.int32, dist.shape, 0)
    idx = jnp.min(jnp.where(dist == dmin, iota, jnp.int32(K)), axis=0,
                  keepdims=True)
    idx_ref[...] = idx


def _encode(xt, w1, b1, w2, b2, w3, b3, w4, b4, et, esq):
    full = lambda *s: pl.BlockSpec(s, lambda i: (0,) * len(s))
    return pl.pallas_call(
        _encode_body,
        grid=(NBE,),
        in_specs=[
            pl.BlockSpec((IN, BBE), lambda i: (0, i)),
            full(60, IN), full(60, 1),
            full(30, 60), full(30, 1),
            full(10, 30), full(10, 1),
            full(10, D), full(D, 1),
            full(D, K), full(K, 1),
        ],
        out_specs=[
            pl.BlockSpec((D, BBE), lambda i: (0, i)),
            pl.BlockSpec((1, BBE), lambda i: (0, i)),
        ],
        out_shape=[
            jax.ShapeDtypeStruct((D, B), jnp.float32),
            jax.ShapeDtypeStruct((1, B), jnp.int32),
        ],
    )(xt, w1, b1, w2, b2, w3, b3, w4, b4, et, esq)


_sc_gather_impl = None


def _build_sc_gather():
    mesh = plsc.VectorSubcoreMesh(core_axis_name="c", subcore_axis_name="s",
                                  num_cores=NC, num_subcores=NS)
    DGN = 4
    BGN = NW // DGN
    DPG = D // DGN
    BPG = B // BGN
    NGRP = BPG // L

    @functools.partial(
        pl.kernel,
        mesh=mesh,
        compiler_params=pltpu.CompilerParams(needs_layout_passes=False),
        out_type=jax.ShapeDtypeStruct((D, B), jnp.float32),
        scratch_types=[
            pltpu.VMEM((DPG * K,), jnp.float32),
            pltpu.VMEM((BPG,), jnp.int32),
            pltpu.VMEM((DPG, BPG), jnp.float32),
        ],
    )
    def body(et_hbm, idx_hbm, out_hbm, et_v, idx_v, zqt_v):
        wid = lax.axis_index("s") * NC + lax.axis_index("c")
        dg = wid // BGN
        bg = wid % BGN
        pltpu.sync_copy(et_hbm.at[pl.ds(dg * DPG * K, DPG * K)], et_v)
        pltpu.sync_copy(idx_hbm.at[pl.ds(bg * BPG, BPG)], idx_v)

        def grp_body(g, carry):
            idxv = idx_v[pl.ds(g * L, L)]
            for dl in range(DPG):
                col = plsc.load_gather(et_v, [idxv + dl * K])
                zqt_v[dl, pl.ds(g * L, L)] = col
            return carry

        lax.fori_loop(0, NGRP, grp_body, 0)
        pltpu.sync_copy(
            zqt_v, out_hbm.at[pl.ds(dg * DPG, DPG), pl.ds(bg * BPG, BPG)])

    return body


def _sc_gather(et_flat, idx):
    global _sc_gather_impl
    if _sc_gather_impl is None:
        _sc_gather_impl = _build_sc_gather()
    return _sc_gather_impl(et_flat, idx)


def _decode_body(xt_ref, zet_ref, zqt_in_ref, w5, b5, w6t, b6, w7t, b7,
                 w8t, b8, w9, b9, w10, b10, xht_ref, zaugt_ref, gt_ref,
                 vq_ref):
    zqt = zqt_in_ref[...]
    tdot = lambda a, b: lax.dot_general(a, b, (((0,), (0,)), ((), ())))
    h = jnp.tanh(jnp.dot(w5[...], zqt) + b5[...])
    h = jnp.tanh(tdot(w6t[...], h) + b6[...])
    h = jnp.tanh(tdot(w7t[...], h) + b7[...])
    xht = tdot(w8t[...], h) + b8[...]
    xht_ref[...] = xht
    xt = xt_ref[...]
    diff = xt - xht
    dn = jnp.sqrt(jnp.sum(diff * diff, axis=0, keepdims=True))
    xn = jnp.sqrt(jnp.sum(xt * xt, axis=0, keepdims=True))
    xhn = jnp.sqrt(jnp.sum(xht * xht, axis=0, keepdims=True))
    rec1 = dn / (xn + 1e-12)
    rec2 = jnp.sum(xt * xht, axis=0, keepdims=True) / (
        jnp.maximum(xn, 1e-8) * jnp.maximum(xhn, 1e-8))
    zaugt = jnp.concatenate([zqt, rec1, rec2], axis=0)
    zaugt_ref[...] = zaugt
    h9 = jnp.tanh(jnp.dot(w9[...], zaugt) + b9[...])
    logits = jnp.dot(w10[...], h9) + b10[...]
    lmax = jnp.max(logits, axis=0, keepdims=True)
    e = jnp.exp(logits - lmax)
    gt_ref[...] = e / jnp.sum(e, axis=0, keepdims=True)
    d = zqt - zet_ref[...]
    s = jnp.sum(d * d)
    i = pl.program_id(0)

    @pl.when(i == 0)
    def _():
        vq_ref[0, 0, 0] = s

    @pl.when(i > 0)
    def _():
        vq_ref[0, 0, 0] += s

    @pl.when(i == NBD - 1)
    def _():
        m = vq_ref[0, 0, 0] / (B * D)
        vq_ref[0, 0, 0] = m + COMMIT * m


def _decode(xt, zet, zqt, w5, b5, w6, b6, w7, b7, w8, b8, w9, b9,
            w10, b10):
    full = lambda *s: pl.BlockSpec(s, lambda i: (0,) * len(s))
    return pl.pallas_call(
        _decode_body,
        grid=(NBD,),
        in_specs=[
            pl.BlockSpec((IN, BBD), lambda i: (0, i)),
            pl.BlockSpec((D, BBD), lambda i: (0, i)),
            pl.BlockSpec((D, BBD), lambda i: (0, i)),
            full(10, D), full(10, 1),
            full(10, 30), full(30, 1),
            full(30, 60), full(60, 1),
            full(60, IN), full(IN, 1),
            full(10, D + 2), full(10, 1),
            full(NG, 10), full(NG, 1),
        ],
        out_specs=[
            pl.BlockSpec((IN, BBD), lambda i: (0, i)),
            pl.BlockSpec((D + 2, BBD), lambda i: (0, i)),
            pl.BlockSpec((NG, BBD), lambda i: (0, i)),
            pl.BlockSpec((1, 1, 1), lambda i: (0, 0, 0),
                         memory_space=pltpu.SMEM),
        ],
        out_shape=[
            jax.ShapeDtypeStruct((IN, B), jnp.float32),
            jax.ShapeDtypeStruct((D + 2, B), jnp.float32),
            jax.ShapeDtypeStruct((NG, B), jnp.float32),
            jax.ShapeDtypeStruct((1, 1, 1), jnp.float32),
        ],
    )(xt, zet, zqt, w5, b5, w6, b6, w7, b7, w8, b8, w9, b9, w10, b10)


def kernel(x, params):
    p = params
    col = lambda b: b.reshape(-1, 1)
    xt = x.T
    et = p['codebook'].T
    esq = jnp.sum(p['codebook'] ** 2, axis=1).reshape(K, 1)

    zet, idx2 = _encode(
        xt, p['W1'], col(p['b1']), p['W2'], col(p['b2']),
        p['W3'], col(p['b3']), p['W4'].T, col(p['b4']), et, esq)

    zqt = _sc_gather(et.reshape(D * K), idx2.reshape(B))

    xht, zaugt, gt, vq_out = _decode(
        xt, zet, zqt, p['W5'], col(p['b5']), p['W6'].T, col(p['b6']),
        p['W7'].T, col(p['b7']), p['W8'].T, col(p['b8']),
        p['W9'], col(p['b9']), p['W10'], col(p['b10']))

    vq_loss = vq_out.reshape(())
    return (zet.T, zqt.T, vq_loss, xht.T, zaugt.T, gt.T)


# speedup vs baseline: 11.1466x; 1.1616x over previous
import functools

import jax
import jax.numpy as jnp
from jax import lax
from jax.experimental import pallas as pl
from jax.experimental.pallas import tpu as pltpu
from jax.experimental.pallas import tpu_sc as plsc

B = 16384
IN = 118
K = 1024
D = 64
NG = 2
COMMIT = 0.25

BBE = 4096
NBE = B // BBE
BBD = 2048
NBD = B // BBD

NC = 2
NS = 16
NW = NC * NS
BPW = B // NW
L = 16


def _encode_body(xt_ref, w1, w2, w3, w4t, et_ref, cst_ref,
                 zet_ref, idx_ref):
    cst = cst_ref[...]
    esq = cst[0:K]
    b1 = cst[K:K + 60]
    b2 = cst[K + 64:K + 64 + 30]
    b3 = cst[K + 128:K + 128 + 10]
    b4 = cst[K + 192:K + 192 + D]
    xt = xt_ref[...]
    h = jnp.tanh(jnp.dot(w1[...], xt) + b1)
    h = jnp.tanh(jnp.dot(w2[...], h) + b2)
    h = jnp.tanh(jnp.dot(w3[...], h) + b3)
    zet = lax.dot_general(w4t[...], h, (((0,), (0,)), ((), ()))) + b4
    zet_ref[...] = zet
    m2 = lax.dot_general(et_ref[...], -2.0 * zet,
                         (((0,), (0,)), ((), ())))
    zsq = jnp.sum(zet * zet, axis=0, keepdims=True)
    dist = (zsq + esq) + m2
    idx_ref[...] = jnp.argmin(dist, axis=0, keepdims=True).astype(jnp.int32)


def _encode(xt, w1, w2, w3, w4, et, cst):
    full = lambda *s: pl.BlockSpec(s, lambda i: (0,) * len(s))
    return pl.pallas_call(
        _encode_body,
        grid=(NBE,),
        in_specs=[
            pl.BlockSpec((IN, BBE), lambda i: (0, i)),
            full(60, IN),
            full(30, 60),
            full(10, 30),
            full(10, D),
            full(D, K), full(K + 256, 1),
        ],
        out_specs=[
            pl.BlockSpec((D, BBE), lambda i: (0, i)),
            pl.BlockSpec((1, BBE), lambda i: (0, i)),
        ],
        out_shape=[
            jax.ShapeDtypeStruct((D, B), jnp.float32),
            jax.ShapeDtypeStruct((1, B), jnp.int32),
        ],
    )(xt, w1, w2, w3, w4, et, cst)


_sc_gather_impl = None


def _build_sc_gather():
    mesh = plsc.VectorSubcoreMesh(core_axis_name="c", subcore_axis_name="s",
                                  num_cores=NC, num_subcores=NS)
    DGN = 4
    BGN = NW // DGN
    DPG = D // DGN
    BPG = B // BGN
    NGRP = BPG // L

    @functools.partial(
        pl.kernel,
        mesh=mesh,
        compiler_params=pltpu.CompilerParams(needs_layout_passes=False),
        out_type=jax.ShapeDtypeStruct((D, B), jnp.float32),
        scratch_types=[
            pltpu.VMEM((DPG, K), jnp.float32),
            pltpu.VMEM((BPG,), jnp.int32),
            pltpu.VMEM((DPG, BPG), jnp.float32),
        ],
    )
    def body(et_hbm, idx_hbm, out_hbm, et_v, idx_v, zqt_v):
        wid = lax.axis_index("s") * NC + lax.axis_index("c")
        dg = wid // BGN
        bg = wid % BGN
        pltpu.sync_copy(et_hbm.at[pl.ds(dg * DPG, DPG)], et_v)
        pltpu.sync_copy(idx_hbm.at[pl.ds(bg * BPG, BPG)], idx_v)

        def grp_body(g, carry):
            idxv = idx_v[pl.ds(g * L, L)]
            for dl in range(DPG):
                dsplat = jnp.full((L,), dl, jnp.int32)
                col = plsc.load_gather(et_v, [dsplat, idxv])
                zqt_v[dl, pl.ds(g * L, L)] = col
            return carry

        lax.fori_loop(0, NGRP, grp_body, 0)
        pltpu.sync_copy(
            zqt_v, out_hbm.at[pl.ds(dg * DPG, DPG), pl.ds(bg * BPG, BPG)])

    return body


def _sc_gather(et_flat, idx):
    global _sc_gather_impl
    if _sc_gather_impl is None:
        _sc_gather_impl = _build_sc_gather()
    return _sc_gather_impl(et_flat, idx)


def _decode_body(xt_ref, zet_ref, zqt_in_ref, w5, w6t, w7t, w8t, w9, w10,
                 cst_ref, zqt_ref, xht_ref, zaugt_ref, gt_ref, vq_ref):
    cst = cst_ref[...]
    b5 = cst[0:10]
    b6 = cst[64:64 + 30]
    b7 = cst[128:128 + 60]
    b8 = cst[192:192 + IN]
    b9 = cst[320:320 + 10]
    b10 = cst[384:384 + NG]
    zqt = zqt_in_ref[...]
    zqt_ref[...] = zqt
    tdot = lambda a, b: lax.dot_general(a, b, (((0,), (0,)), ((), ())))
    h = jnp.tanh(jnp.dot(w5[...], zqt) + b5)
    h = jnp.tanh(tdot(w6t[...], h) + b6)
    h = jnp.tanh(tdot(w7t[...], h) + b7)
    xht = tdot(w8t[...], h) + b8
    xht_ref[...] = xht
    xt = xt_ref[...]
    diff = xt - xht
    dn = jnp.sqrt(jnp.sum(diff * diff, axis=0, keepdims=True))
    xn = jnp.sqrt(jnp.sum(xt * xt, axis=0, keepdims=True))
    xhn = jnp.sqrt(jnp.sum(xht * xht, axis=0, keepdims=True))
    rec1 = dn / (xn + 1e-12)
    rec2 = jnp.sum(xt * xht, axis=0, keepdims=True) / (
        jnp.maximum(xn, 1e-8) * jnp.maximum(xhn, 1e-8))
    zaugt = jnp.concatenate([zqt, rec1, rec2], axis=0)
    zaugt_ref[...] = zaugt
    h9 = jnp.tanh(jnp.dot(w9[...], zaugt) + b9)
    logits = jnp.dot(w10[...], h9) + b10
    lmax = jnp.max(logits, axis=0, keepdims=True)
    e = jnp.exp(logits - lmax)
    gt_ref[...] = e / jnp.sum(e, axis=0, keepdims=True)
    d = zqt - zet_ref[...]
    s = jnp.sum(d * d)
    i = pl.program_id(0)

    @pl.when(i == 0)
    def _():
        vq_ref[0, 0, 0] = s

    @pl.when(i > 0)
    def _():
        vq_ref[0, 0, 0] += s

    @pl.when(i == NBD - 1)
    def _():
        m = vq_ref[0, 0, 0] / (B * D)
        vq_ref[0, 0, 0] = m + COMMIT * m


def _decode(xt, zet, zqt, w5, w6, w7, w8, w9, w10, cst):
    full = lambda *s: pl.BlockSpec(s, lambda i: (0,) * len(s))
    return pl.pallas_call(
        _decode_body,
        grid=(NBD,),
        in_specs=[
            pl.BlockSpec((IN, BBD), lambda i: (0, i)),
            pl.BlockSpec((D, BBD), lambda i: (0, i)),
            pl.BlockSpec((D, BBD), lambda i: (0, i)),
            full(10, D),
            full(10, 30),
            full(30, 60),
            full(60, IN),
            full(10, D + 2),
            full(NG, 10),
            full(386, 1),
        ],
        out_specs=[
            pl.BlockSpec((D, BBD), lambda i: (0, i)),
            pl.BlockSpec((IN, BBD), lambda i: (0, i)),
            pl.BlockSpec((D + 2, BBD), lambda i: (0, i)),
            pl.BlockSpec((NG, BBD), lambda i: (0, i)),
            pl.BlockSpec((1, 1, 1), lambda i: (0, 0, 0),
                         memory_space=pltpu.SMEM),
        ],
        out_shape=[
            jax.ShapeDtypeStruct((D, B), jnp.float32),
            jax.ShapeDtypeStruct((IN, B), jnp.float32),
            jax.ShapeDtypeStruct((D + 2, B), jnp.float32),
            jax.ShapeDtypeStruct((NG, B), jnp.float32),
            jax.ShapeDtypeStruct((1, 1, 1), jnp.float32),
        ],
    )(xt, zet, zqt, w5, w6, w7, w8, w9, w10, cst)


def kernel(x, params):
    p = params
    xt = x.T
    et = p['codebook'].T
    esq = jnp.sum(p['codebook'] ** 2, axis=1)

    enc_cst = jnp.concatenate([
        esq,
        jnp.pad(p['b1'], (0, 4)), jnp.pad(p['b2'], (0, 34)),
        jnp.pad(p['b3'], (0, 54)), p['b4'],
    ]).reshape(K + 256, 1)
    dec_cst = jnp.concatenate([
        jnp.pad(p['b5'], (0, 54)), jnp.pad(p['b6'], (0, 34)),
        jnp.pad(p['b7'], (0, 4)), jnp.pad(p['b8'], (0, 10)),
        jnp.pad(p['b9'], (0, 54)), p['b10'],
    ]).reshape(386, 1)

    zet, idx2 = _encode(xt, p['W1'], p['W2'], p['W3'], p['W4'].T, et,
                        enc_cst)

    zqt_sc = _sc_gather(et, idx2.reshape(B))

    zqt, xht, zaugt, gt, vq_out = _decode(
        xt, zet, zqt_sc, p['W5'], p['W6'].T, p['W7'].T, p['W8'].T,
        p['W9'], p['W10'], dec_cst)

    vq_loss = vq_out.reshape(())
    return (zet.T, zqt.T, vq_loss, xht.T, zaugt.T, gt.T)


# speedup vs baseline: 11.7644x; 1.0554x over previous
import functools

import jax
import jax.numpy as jnp
from jax import lax
from jax.experimental import pallas as pl
from jax.experimental.pallas import tpu as pltpu
from jax.experimental.pallas import tpu_sc as plsc

B = 16384
IN = 118
K = 1024
D = 64
NG = 2
COMMIT = 0.25

BBE = 4096
NBE = B // BBE
BBD = 4096
NBD = B // BBD

NC = 2
NS = 16
NW = NC * NS
BPW = B // NW
L = 16


def _encode_body(xt_ref, w1, w2, w3, w4t, et_ref, cst_ref,
                 zet_ref, idx_ref):
    cst = cst_ref[...]
    esq = cst[0:K]
    b1 = cst[K:K + 60]
    b2 = cst[K + 64:K + 64 + 30]
    b3 = cst[K + 128:K + 128 + 10]
    b4 = cst[K + 192:K + 192 + D]
    xt = xt_ref[...]
    h = jnp.tanh(jnp.dot(w1[...], xt) + b1)
    h = jnp.tanh(jnp.dot(w2[...], h) + b2)
    h = jnp.tanh(jnp.dot(w3[...], h) + b3)
    zet = lax.dot_general(w4t[...], h, (((0,), (0,)), ((), ()))) + b4
    zet_ref[...] = zet
    m2 = lax.dot_general(et_ref[...], -2.0 * zet,
                         (((0,), (0,)), ((), ())))
    zsq = jnp.sum(zet * zet, axis=0, keepdims=True)
    dist = (zsq + esq) + m2
    idx_ref[...] = jnp.argmin(dist, axis=0, keepdims=True).astype(jnp.int32)


def _encode(xt, w1, w2, w3, w4, et, cst):
    full = lambda *s: pl.BlockSpec(s, lambda i: (0,) * len(s))
    return pl.pallas_call(
        _encode_body,
        grid=(NBE,),
        in_specs=[
            pl.BlockSpec((IN, BBE), lambda i: (0, i)),
            full(60, IN),
            full(30, 60),
            full(10, 30),
            full(10, D),
            full(D, K), full(K + 256, 1),
        ],
        out_specs=[
            pl.BlockSpec((D, BBE), lambda i: (0, i)),
            pl.BlockSpec((1, BBE), lambda i: (0, i)),
        ],
        out_shape=[
            jax.ShapeDtypeStruct((D, B), jnp.float32),
            jax.ShapeDtypeStruct((1, B), jnp.int32),
        ],
    )(xt, w1, w2, w3, w4, et, cst)


_sc_gather_impl = None


def _build_sc_gather():
    mesh = plsc.VectorSubcoreMesh(core_axis_name="c", subcore_axis_name="s",
                                  num_cores=NC, num_subcores=NS)
    DGN = 4
    BGN = NW // DGN
    DPG = D // DGN
    BPG = B // BGN
    NGRP = BPG // L

    @functools.partial(
        pl.kernel,
        mesh=mesh,
        compiler_params=pltpu.CompilerParams(needs_layout_passes=False,
                                             skip_device_barrier=True),
        out_type=jax.ShapeDtypeStruct((D, B), jnp.float32),
        scratch_types=[
            pltpu.VMEM((DPG, K), jnp.float32),
            pltpu.VMEM((BPG,), jnp.int32),
            pltpu.VMEM((DPG, BPG), jnp.float32),
        ],
    )
    def body(et_hbm, idx_hbm, out_hbm, et_v, idx_v, zqt_v):
        wid = lax.axis_index("s") * NC + lax.axis_index("c")
        dg = wid // BGN
        bg = wid % BGN
        pltpu.sync_copy(et_hbm.at[pl.ds(dg * DPG, DPG)], et_v)
        pltpu.sync_copy(idx_hbm.at[pl.ds(bg * BPG, BPG)], idx_v)

        def grp_body(g, carry):
            idxv = idx_v[pl.ds(g * L, L)]
            for dl in range(DPG):
                dsplat = jnp.full((L,), dl, jnp.int32)
                col = plsc.load_gather(et_v, [dsplat, idxv])
                zqt_v[dl, pl.ds(g * L, L)] = col
            return carry

        lax.fori_loop(0, NGRP, grp_body, 0, unroll=4)
        pltpu.sync_copy(
            zqt_v, out_hbm.at[pl.ds(dg * DPG, DPG), pl.ds(bg * BPG, BPG)])

    return body


def _sc_gather(et_flat, idx):
    global _sc_gather_impl
    if _sc_gather_impl is None:
        _sc_gather_impl = _build_sc_gather()
    return _sc_gather_impl(et_flat, idx)


def _decode_body(xt_ref, zet_ref, zqt_in_ref, w5, w6t, w7t, w8t, w9, w10,
                 cst_ref, zqt_ref, xht_ref, zaugt_ref, gt_ref, vq_ref):
    cst = cst_ref[...]
    b5 = cst[0:10]
    b6 = cst[64:64 + 30]
    b7 = cst[128:128 + 60]
    b8 = cst[192:192 + IN]
    b9 = cst[320:320 + 10]
    b10 = cst[384:384 + NG]
    zqt = zqt_in_ref[...]
    zqt_ref[...] = zqt
    tdot = lambda a, b: lax.dot_general(a, b, (((0,), (0,)), ((), ())))
    h = jnp.tanh(jnp.dot(w5[...], zqt) + b5)
    h = jnp.tanh(tdot(w6t[...], h) + b6)
    h = jnp.tanh(tdot(w7t[...], h) + b7)
    xht = tdot(w8t[...], h) + b8
    xht_ref[...] = xht
    xt = xt_ref[...]
    diff = xt - xht
    dn = jnp.sqrt(jnp.sum(diff * diff, axis=0, keepdims=True))
    xn = jnp.sqrt(jnp.sum(xt * xt, axis=0, keepdims=True))
    xhn = jnp.sqrt(jnp.sum(xht * xht, axis=0, keepdims=True))
    rec1 = dn / (xn + 1e-12)
    rec2 = jnp.sum(xt * xht, axis=0, keepdims=True) / (
        jnp.maximum(xn, 1e-8) * jnp.maximum(xhn, 1e-8))
    zaugt = jnp.concatenate([zqt, rec1, rec2], axis=0)
    zaugt_ref[...] = zaugt
    h9 = jnp.tanh(jnp.dot(w9[...], zaugt) + b9)
    logits = jnp.dot(w10[...], h9) + b10
    lmax = jnp.max(logits, axis=0, keepdims=True)
    e = jnp.exp(logits - lmax)
    gt_ref[...] = e / jnp.sum(e, axis=0, keepdims=True)
    d = zqt - zet_ref[...]
    s = jnp.sum(d * d)
    i = pl.program_id(0)

    @pl.when(i == 0)
    def _():
        vq_ref[0, 0, 0] = s

    @pl.when(i > 0)
    def _():
        vq_ref[0, 0, 0] += s

    @pl.when(i == NBD - 1)
    def _():
        m = vq_ref[0, 0, 0] / (B * D)
        vq_ref[0, 0, 0] = m + COMMIT * m


def _decode(xt, zet, zqt, w5, w6, w7, w8, w9, w10, cst):
    full = lambda *s: pl.BlockSpec(s, lambda i: (0,) * len(s))
    return pl.pallas_call(
        _decode_body,
        grid=(NBD,),
        in_specs=[
            pl.BlockSpec((IN, BBD), lambda i: (0, i)),
            pl.BlockSpec((D, BBD), lambda i: (0, i)),
            pl.BlockSpec((D, BBD), lambda i: (0, i)),
            full(10, D),
            full(10, 30),
            full(30, 60),
            full(60, IN),
            full(10, D + 2),
            full(NG, 10),
            full(386, 1),
        ],
        out_specs=[
            pl.BlockSpec((D, BBD), lambda i: (0, i)),
            pl.BlockSpec((IN, BBD), lambda i: (0, i)),
            pl.BlockSpec((D + 2, BBD), lambda i: (0, i)),
            pl.BlockSpec((NG, BBD), lambda i: (0, i)),
            pl.BlockSpec((1, 1, 1), lambda i: (0, 0, 0),
                         memory_space=pltpu.SMEM),
        ],
        out_shape=[
            jax.ShapeDtypeStruct((D, B), jnp.float32),
            jax.ShapeDtypeStruct((IN, B), jnp.float32),
            jax.ShapeDtypeStruct((D + 2, B), jnp.float32),
            jax.ShapeDtypeStruct((NG, B), jnp.float32),
            jax.ShapeDtypeStruct((1, 1, 1), jnp.float32),
        ],
    )(xt, zet, zqt, w5, w6, w7, w8, w9, w10, cst)


def kernel(x, params):
    p = params
    xt = x.T
    et = p['codebook'].T
    esq = jnp.sum(p['codebook'] ** 2, axis=1)

    enc_cst = jnp.concatenate([
        esq,
        jnp.pad(p['b1'], (0, 4)), jnp.pad(p['b2'], (0, 34)),
        jnp.pad(p['b3'], (0, 54)), p['b4'],
    ]).reshape(K + 256, 1)
    dec_cst = jnp.concatenate([
        jnp.pad(p['b5'], (0, 54)), jnp.pad(p['b6'], (0, 34)),
        jnp.pad(p['b7'], (0, 4)), jnp.pad(p['b8'], (0, 10)),
        jnp.pad(p['b9'], (0, 54)), p['b10'],
    ]).reshape(386, 1)

    zet, idx2 = _encode(xt, p['W1'], p['W2'], p['W3'], p['W4'].T, et,
                        enc_cst)

    zqt_sc = _sc_gather(et, idx2.reshape(B))

    zqt, xht, zaugt, gt, vq_out = _decode(
        xt, zet, zqt_sc, p['W5'], p['W6'].T, p['W7'].T, p['W8'].T,
        p['W9'], p['W10'], dec_cst)

    vq_loss = vq_out.reshape(())
    return (zet.T, zqt.T, vq_loss, xht.T, zaugt.T, gt.T)
